# Initial kernel scaffold; baseline (speedup 1.0000x reference)
import jax
import jax.numpy as jnp
from jax.experimental import pallas as pl


def kernel(x_user, x_item, edge_u2i, edge_i2u, Wemb_u, bemb_u, Wemb_i, bemb_i, Wself_u, Wself_i, bias_u, bias_i, Wmsg_u2i, Wmsg_i2u, Wmlp, bmlp):
    raise NotImplementedError("write your pallas kernel here")


# speedup vs baseline: 2.0088x; 2.0088x over previous
import functools

import jax
import jax.numpy as jnp
from jax import lax
from jax.experimental import pallas as pl
from jax.experimental.pallas import tpu as pltpu
from jax.experimental.pallas import tpu_sc as plsc

f32 = jnp.float32
i32 = jnp.int32

NU = 10000
NI = 10000
NP = 10240
NTOT = 2 * NP
E = 160000
NCOLS = 4
HID = 256
HALF = 128
OUTD = 64

NS = 16
CHUNK = 128
CPP = 1280
EPAD = CPP * CHUNK
CPS = CPP // NS
ACC_ROWS = NP
TRASH = 10100
ZROWS = ACC_ROWS // NS
OROWS = ZROWS

RB = 640
NB_U = NP // RB


def _a0_body(x_ref, ws_ref, bs_ref, wm_ref, bm_ref, su_ref, ml_ref, mr_ref):
    x = x_ref[...]
    su_ref[...] = jnp.dot(x, ws_ref[0], preferred_element_type=f32) + bs_ref[0]
    m = jnp.dot(x, wm_ref[0], preferred_element_type=f32) + bm_ref[0]
    ml_ref[...] = m[:, :HALF]
    mr_ref[...] = m[:, HALF:]


def _mid_body(su_ref, ml_ref, mr_ref, ws_ref, bs_ref, wm_ref, su_o, ml_o, mr_o):
    h = su_ref[...] + jnp.concatenate([ml_ref[...], mr_ref[...]], axis=1)
    h = jnp.maximum(h, 0.0)
    su_o[...] = jnp.dot(h, ws_ref[0], preferred_element_type=f32) + bs_ref[0]
    m = jnp.dot(h, wm_ref[0], preferred_element_type=f32)
    ml_o[...] = m[:, :HALF]
    mr_o[...] = m[:, HALF:]


def _fuse_body(su_ref, ml_ref, mr_ref, w_ref, b_ref, o_ref):
    h = su_ref[...] + jnp.concatenate([ml_ref[...], mr_ref[...]], axis=1)
    h = jnp.maximum(h, 0.0)
    o_ref[...] = jnp.dot(h, w_ref[0], preferred_element_type=f32) + b_ref[0]


def _fuse_msg_body(su_ref, ml_ref, mr_ref, w_ref, ml_o, mr_o):
    h = su_ref[...] + jnp.concatenate([ml_ref[...], mr_ref[...]], axis=1)
    h = jnp.maximum(h, 0.0)
    m = jnp.dot(h, w_ref[0], preferred_element_type=f32)
    ml_o[...] = m[:, :HALF]
    mr_o[...] = m[:, HALF:]


def _rows(cols, off=0):
    return pl.BlockSpec((RB, cols), lambda i, o=off: (i + o, 0))


def _wspec(a, b):
    return pl.BlockSpec((1, a, b), lambda i: (i // NB_U, 0, 0))


def _bspec(b):
    return pl.BlockSpec((1, 1, b), lambda i: (i // NB_U, 0, 0))


def _wfix(a, b):
    return pl.BlockSpec((1, a, b), lambda i: (0, 0, 0))


def _bfix(b):
    return pl.BlockSpec((1, 1, b), lambda i: (0, 0, 0))


def _sds(r, c):
    return jax.ShapeDtypeStruct((r, c), f32)


def _a0_call(x, ws, bs, wm, bm):
    return pl.pallas_call(
        _a0_body,
        grid=(2 * NB_U,),
        in_specs=[_rows(NCOLS), _wspec(NCOLS, HID), _bspec(HID),
                  _wspec(NCOLS, HID), _bspec(HID)],
        out_specs=[_rows(HID), _rows(HALF), _rows(HALF)],
        out_shape=[_sds(NTOT, HID), _sds(NTOT, HALF), _sds(NTOT, HALF)],
    )(x, ws, bs, wm, bm)


def _mid_call(su, ml, mr, ws, bs, wm):
    return pl.pallas_call(
        _mid_body,
        grid=(2 * NB_U,),
        in_specs=[_rows(HID), _rows(HALF), _rows(HALF),
                  _wspec(HID, HID), _bspec(HID), _wspec(HID, HID)],
        out_specs=[_rows(HID), _rows(HALF), _rows(HALF)],
        out_shape=[_sds(NTOT, HID), _sds(NTOT, HALF), _sds(NTOT, HALF)],
    )(su, ml, mr, ws, bs, wm)


def _fuse_call(su, ml, mr, w, b, off, outc):
    return pl.pallas_call(
        _fuse_body,
        grid=(NB_U,),
        in_specs=[_rows(HID, off), _rows(HALF, off), _rows(HALF, off),
                  _wfix(HID, outc), _bfix(outc)],
        out_specs=_rows(outc),
        out_shape=_sds(NP, outc),
    )(su, ml, mr, w, b)


def _fuse_msg_call(su, ml, mr, w, off):
    return pl.pallas_call(
        _fuse_msg_body,
        grid=(NB_U,),
        in_specs=[_rows(HID, off), _rows(HALF, off), _rows(HALF, off),
                  _wfix(HID, HID)],
        out_specs=[_rows(HALF), _rows(HALF)],
        out_shape=[_sds(NP, HALF), _sds(NP, HALF)],
    )(su, ml, mr, w)


def _make_seg(nphase):
    out_rows = nphase * NP
    mesh = plsc.VectorSubcoreMesh(core_axis_name="c", subcore_axis_name="s")

    @functools.partial(
        pl.kernel,
        out_type=(jax.ShapeDtypeStruct((out_rows, HALF), f32),
                  jax.ShapeDtypeStruct((out_rows, HALF), f32)),
        mesh=mesh,
        scratch_types=(
            pltpu.VMEM((CPS, CHUNK), i32),
            pltpu.VMEM((CPS, CHUNK), i32),
            pltpu.VMEM((CHUNK, HALF), f32),
            pltpu.VMEM_SHARED((ACC_ROWS, HALF), f32),
            pltpu.SemaphoreType.DMA,
        ),
    )
    def seg(ml, mr, srcs, dsts, zeros, outl, outr, srcv, dstv, buf, accum, gsem):
        c = lax.axis_index("c")
        s = lax.axis_index("s")

        def run(tab, out):
            for p in range(nphase):
                pltpu.sync_copy(zeros, accum.at[pl.ds(s * ZROWS, ZROWS), :])
                pltpu.sync_copy(srcs.at[pl.ds(p * CPP + s * CPS, CPS), :], srcv)
                pltpu.sync_copy(dsts.at[pl.ds(p * CPP + s * CPS, CPS), :], dstv)
                plsc.subcore_barrier()

                def step(g, _):
                    pltpu.async_copy(tab.at[srcv.at[g]], buf, gsem).wait()
                    pltpu.sync_copy(buf, accum.at[dstv.at[g]], add=True)
                    return 0

                lax.fori_loop(0, CPS, step, 0)
                plsc.subcore_barrier()
                pltpu.sync_copy(accum.at[pl.ds(s * OROWS, OROWS), :],
                                out.at[pl.ds(p * NP + s * OROWS, OROWS), :])
                plsc.subcore_barrier()

        @pl.when(c == 0)
        def _():
            run(ml, outl)

        @pl.when(c == 1)
        def _():
            run(mr, outr)

    return seg


def kernel(x_user, x_item, edge_u2i, edge_i2u, Wemb_u, bemb_u, Wemb_i, bemb_i,
           Wself_u, Wself_i, bias_u, bias_i, Wmsg_u2i, Wmsg_i2u, Wmlp, bmlp):
    eye = jnp.eye(NCOLS, dtype=f32)
    Wbig_u = (eye[:, :, None] * Wemb_u[None]).reshape(NCOLS, HID)
    Wbig_i = (eye[:, :, None] * Wemb_i[None]).reshape(NCOLS, HID)
    bflat_u = bemb_u.reshape(HID)
    bflat_i = bemb_i.reshape(HID)
    WsF = jnp.stack([Wbig_u @ Wself_u[0], Wbig_i @ Wself_i[0]])
    bsF = jnp.stack([bflat_u @ Wself_u[0] + bias_u[0],
                     bflat_i @ Wself_i[0] + bias_i[0]])[:, None, :]
    WmF = jnp.stack([Wbig_u @ Wmsg_u2i[0], Wbig_i @ Wmsg_i2u[0]])
    bmF = jnp.stack([bflat_u @ Wmsg_u2i[0], bflat_i @ Wmsg_i2u[0]])[:, None, :]
    Ws1 = jnp.stack([Wself_u[1], Wself_i[1]])
    bs1 = jnp.stack([bias_u[1], bias_i[1]])[:, None, :]
    Wm1 = jnp.stack([Wmsg_u2i[1], Wmsg_i2u[1]])

    src_u2i = edge_u2i[0].astype(i32)
    dst_u2i = edge_u2i[1].astype(i32)
    src_i2u = edge_i2u[0].astype(i32)
    dst_i2u = edge_i2u[1].astype(i32)
    pad_s = jnp.zeros((EPAD - E,), i32)
    pad_d = jnp.full((EPAD - E,), TRASH, i32)
    SRC2 = jnp.concatenate([src_i2u + NP, pad_s, src_u2i, pad_s]).reshape(2 * CPP, CHUNK)
    DST2 = jnp.concatenate([dst_i2u, pad_d, dst_u2i, pad_d]).reshape(2 * CPP, CHUNK)
    SRC1 = jnp.concatenate([src_i2u, pad_s]).reshape(CPP, CHUNK)
    DST1 = jnp.concatenate([dst_i2u, pad_d]).reshape(CPP, CHUNK)
    zeros = jnp.zeros((ZROWS, HALF), f32)

    X = jnp.concatenate([jnp.pad(x_user, ((0, NP - NU), (0, 0))),
                         jnp.pad(x_item, ((0, NP - NI), (0, 0)))], axis=0)

    SU, ML, MR = _a0_call(X, WsF, bsF, WmF, bmF)
    seg2 = _make_seg(2)
    MSGL, MSGR = seg2(ML, MR, SRC2, DST2, zeros)
    SU, ML, MR = _mid_call(SU, MSGL, MSGR, Ws1, bs1, Wm1)
    MSGL, MSGR = seg2(ML, MR, SRC2, DST2, zeros)
    SU2 = _fuse_call(SU, MSGL, MSGR, Wself_u[2][None], bias_u[2][None, None], 0, HID)
    M2L, M2R = _fuse_msg_call(SU, MSGL, MSGR, Wmsg_i2u[2][None], NB_U)
    seg1 = _make_seg(1)
    MSG2L, MSG2R = seg1(M2L, M2R, SRC1, DST1, zeros)
    out = _fuse_call(SU2, MSG2L, MSG2R, Wmlp[None], bmlp[None, None], 0, OUTD)
    return out[:NU]


# speedup vs baseline: 2.2270x; 1.1086x over previous
import functools

import jax
import jax.numpy as jnp
from jax import lax
from jax.experimental import pallas as pl
from jax.experimental.pallas import tpu as pltpu
from jax.experimental.pallas import tpu_sc as plsc

f32 = jnp.float32
i32 = jnp.int32

NU = 10000
NI = 10000
NP = 10240
NTOT = 2 * NP
E = 160000
NCOLS = 4
HID = 256
HALF = 128
OUTD = 64

NS = 16
CHUNK = 128
CPP = 1280
EPAD = CPP * CHUNK
CPS = CPP // NS
ACC_ROWS = NP
TRASH = 10100
ZROWS = ACC_ROWS // NS
OROWS = ZROWS
NBUF = 2
WIN = 16
NWIN = CPS // WIN

RB = 640
NB_U = NP // RB


def _a0_body(x_ref, ws_ref, bs_ref, wm_ref, bm_ref, su_ref, ml_ref, mr_ref):
    x = x_ref[...]
    su_ref[...] = jnp.dot(x, ws_ref[0], preferred_element_type=f32) + bs_ref[0]
    m = jnp.dot(x, wm_ref[0], preferred_element_type=f32) + bm_ref[0]
    ml_ref[...] = m[:, :HALF]
    mr_ref[...] = m[:, HALF:]


def _mid_body(su_ref, ml_ref, mr_ref, ws_ref, bs_ref, wm_ref, su_o, ml_o, mr_o):
    h = su_ref[...] + jnp.concatenate([ml_ref[...], mr_ref[...]], axis=1)
    h = jnp.maximum(h, 0.0)
    su_o[...] = jnp.dot(h, ws_ref[0], preferred_element_type=f32) + bs_ref[0]
    m = jnp.dot(h, wm_ref[0], preferred_element_type=f32)
    ml_o[...] = m[:, :HALF]
    mr_o[...] = m[:, HALF:]


def _fuse_body(su_ref, ml_ref, mr_ref, w_ref, b_ref, o_ref):
    h = su_ref[...] + jnp.concatenate([ml_ref[...], mr_ref[...]], axis=1)
    h = jnp.maximum(h, 0.0)
    o_ref[...] = jnp.dot(h, w_ref[0], preferred_element_type=f32) + b_ref[0]


def _fuse_msg_body(su_ref, ml_ref, mr_ref, w_ref, ml_o, mr_o):
    h = su_ref[...] + jnp.concatenate([ml_ref[...], mr_ref[...]], axis=1)
    h = jnp.maximum(h, 0.0)
    m = jnp.dot(h, w_ref[0], preferred_element_type=f32)
    ml_o[...] = m[:, :HALF]
    mr_o[...] = m[:, HALF:]


def _rows(cols, off=0):
    return pl.BlockSpec((RB, cols), lambda i, o=off: (i + o, 0))


def _wspec(a, b):
    return pl.BlockSpec((1, a, b), lambda i: (i // NB_U, 0, 0))


def _bspec(b):
    return pl.BlockSpec((1, 1, b), lambda i: (i // NB_U, 0, 0))


def _wfix(a, b):
    return pl.BlockSpec((1, a, b), lambda i: (0, 0, 0))


def _bfix(b):
    return pl.BlockSpec((1, 1, b), lambda i: (0, 0, 0))


def _sds(r, c):
    return jax.ShapeDtypeStruct((r, c), f32)


def _a0_call(x, ws, bs, wm, bm):
    return pl.pallas_call(
        _a0_body,
        grid=(2 * NB_U,),
        in_specs=[_rows(NCOLS), _wspec(NCOLS, HID), _bspec(HID),
                  _wspec(NCOLS, HID), _bspec(HID)],
        out_specs=[_rows(HID), _rows(HALF), _rows(HALF)],
        out_shape=[_sds(NTOT, HID), _sds(NTOT, HALF), _sds(NTOT, HALF)],
    )(x, ws, bs, wm, bm)


def _mid_call(su, ml, mr, ws, bs, wm):
    return pl.pallas_call(
        _mid_body,
        grid=(2 * NB_U,),
        in_specs=[_rows(HID), _rows(HALF), _rows(HALF),
                  _wspec(HID, HID), _bspec(HID), _wspec(HID, HID)],
        out_specs=[_rows(HID), _rows(HALF), _rows(HALF)],
        out_shape=[_sds(NTOT, HID), _sds(NTOT, HALF), _sds(NTOT, HALF)],
    )(su, ml, mr, ws, bs, wm)


def _fuse_call(su, ml, mr, w, b, off, outc):
    return pl.pallas_call(
        _fuse_body,
        grid=(NB_U,),
        in_specs=[_rows(HID, off), _rows(HALF, off), _rows(HALF, off),
                  _wfix(HID, outc), _bfix(outc)],
        out_specs=_rows(outc),
        out_shape=_sds(NP, outc),
    )(su, ml, mr, w, b)


def _fuse_msg_call(su, ml, mr, w, off):
    return pl.pallas_call(
        _fuse_msg_body,
        grid=(NB_U,),
        in_specs=[_rows(HID, off), _rows(HALF, off), _rows(HALF, off),
                  _wfix(HID, HID)],
        out_specs=[_rows(HALF), _rows(HALF)],
        out_shape=[_sds(NP, HALF), _sds(NP, HALF)],
    )(su, ml, mr, w)


def _make_seg(nphase):
    out_rows = nphase * NP
    mesh = plsc.VectorSubcoreMesh(core_axis_name="c", subcore_axis_name="s")

    @functools.partial(
        pl.kernel,
        out_type=(jax.ShapeDtypeStruct((out_rows, HALF), f32),
                  jax.ShapeDtypeStruct((out_rows, HALF), f32)),
        mesh=mesh,
        scratch_types=(
            pltpu.VMEM((2, WIN, CHUNK), i32),
            pltpu.VMEM((2, WIN, CHUNK), i32),
            pltpu.VMEM((NBUF, CHUNK, HALF), f32),
            pltpu.VMEM_SHARED((ACC_ROWS, HALF), f32),
            pltpu.SemaphoreType.DMA((NBUF,)),
            pltpu.SemaphoreType.DMA((NBUF,)),
        ),
    )
    def seg(ml, mr, srcs, dsts, zeros, outl, outr, srcv, dstv, bufs, accum,
            gsem, ssem):
        c = lax.axis_index("c")
        s = lax.axis_index("s")

        def run(tab, out):
            for p in range(nphase):
                base = p * CPP + s * CPS
                pltpu.sync_copy(zeros, accum.at[pl.ds(s * ZROWS, ZROWS), :])
                pltpu.sync_copy(srcs.at[pl.ds(base, WIN), :], srcv.at[0])
                pltpu.sync_copy(dsts.at[pl.ds(base, WIN), :], dstv.at[0])
                pltpu.sync_copy(srcs.at[pl.ds(base + WIN, WIN), :], srcv.at[1])
                pltpu.sync_copy(dsts.at[pl.ds(base + WIN, WIN), :], dstv.at[1])
                plsc.subcore_barrier()

                for b in range(NBUF):
                    pltpu.async_copy(tab.at[srcv.at[0, b]], bufs.at[b],
                                     gsem.at[b])

                def group(i0, _):
                    g0 = i0 * NBUF
                    w = g0 // WIN

                    @pl.when((lax.rem(g0, WIN) == 0) & (w >= 1)
                             & (w + 1 < NWIN))
                    def _():
                        sl = lax.rem(w + 1, 2)
                        pltpu.sync_copy(
                            srcs.at[pl.ds(base + (w + 1) * WIN, WIN), :],
                            srcv.at[sl])
                        pltpu.sync_copy(
                            dsts.at[pl.ds(base + (w + 1) * WIN, WIN), :],
                            dstv.at[sl])

                    for b in range(NBUF):
                        g = g0 + b
                        sl = lax.rem(g // WIN, 2)
                        row = lax.rem(g, WIN)
                        pltpu.make_async_copy(tab.at[srcv.at[sl, row]],
                                              bufs.at[b], gsem.at[b]).wait()
                        pltpu.async_copy(bufs.at[b],
                                         accum.at[dstv.at[sl, row]],
                                         ssem.at[b], add=True)
                    for b in range(NBUF):
                        g = g0 + b
                        g2 = g + NBUF
                        sl = lax.rem(g // WIN, 2)
                        row = lax.rem(g, WIN)
                        sl2 = lax.rem(g2 // WIN, 2)
                        row2 = lax.rem(g2, WIN)
                        pltpu.make_async_copy(bufs.at[b],
                                              accum.at[dstv.at[sl, row]],
                                              ssem.at[b]).wait()

                        @pl.when(g2 < CPS)
                        def _():
                            pltpu.async_copy(tab.at[srcv.at[sl2, row2]],
                                             bufs.at[b], gsem.at[b])
                    return 0

                lax.fori_loop(0, CPS // NBUF, group, 0)
                plsc.subcore_barrier()
                pltpu.sync_copy(accum.at[pl.ds(s * OROWS, OROWS), :],
                                out.at[pl.ds(p * NP + s * OROWS, OROWS), :])
                plsc.subcore_barrier()

        @pl.when(c == 0)
        def _():
            run(ml, outl)

        @pl.when(c == 1)
        def _():
            run(mr, outr)

    return seg


def kernel(x_user, x_item, edge_u2i, edge_i2u, Wemb_u, bemb_u, Wemb_i, bemb_i,
           Wself_u, Wself_i, bias_u, bias_i, Wmsg_u2i, Wmsg_i2u, Wmlp, bmlp):
    eye = jnp.eye(NCOLS, dtype=f32)
    Wbig_u = (eye[:, :, None] * Wemb_u[None]).reshape(NCOLS, HID)
    Wbig_i = (eye[:, :, None] * Wemb_i[None]).reshape(NCOLS, HID)
    bflat_u = bemb_u.reshape(HID)
    bflat_i = bemb_i.reshape(HID)
    WsF = jnp.stack([Wbig_u @ Wself_u[0], Wbig_i @ Wself_i[0]])
    bsF = jnp.stack([bflat_u @ Wself_u[0] + bias_u[0],
                     bflat_i @ Wself_i[0] + bias_i[0]])[:, None, :]
    WmF = jnp.stack([Wbig_u @ Wmsg_u2i[0], Wbig_i @ Wmsg_i2u[0]])
    bmF = jnp.stack([bflat_u @ Wmsg_u2i[0], bflat_i @ Wmsg_i2u[0]])[:, None, :]
    Ws1 = jnp.stack([Wself_u[1], Wself_i[1]])
    bs1 = jnp.stack([bias_u[1], bias_i[1]])[:, None, :]
    Wm1 = jnp.stack([Wmsg_u2i[1], Wmsg_i2u[1]])

    src_u2i = edge_u2i[0].astype(i32)
    dst_u2i = edge_u2i[1].astype(i32)
    src_i2u = edge_i2u[0].astype(i32)
    dst_i2u = edge_i2u[1].astype(i32)
    pad_s = jnp.zeros((EPAD - E,), i32)
    pad_d = jnp.full((EPAD - E,), TRASH, i32)
    SRC2 = jnp.concatenate([src_i2u + NP, pad_s, src_u2i, pad_s]).reshape(2 * CPP, CHUNK)
    DST2 = jnp.concatenate([dst_i2u, pad_d, dst_u2i, pad_d]).reshape(2 * CPP, CHUNK)
    SRC1 = jnp.concatenate([src_i2u, pad_s]).reshape(CPP, CHUNK)
    DST1 = jnp.concatenate([dst_i2u, pad_d]).reshape(CPP, CHUNK)
    zeros = jnp.zeros((ZROWS, HALF), f32)

    X = jnp.concatenate([jnp.pad(x_user, ((0, NP - NU), (0, 0))),
                         jnp.pad(x_item, ((0, NP - NI), (0, 0)))], axis=0)

    SU, ML, MR = _a0_call(X, WsF, bsF, WmF, bmF)
    seg2 = _make_seg(2)
    MSGL, MSGR = seg2(ML, MR, SRC2, DST2, zeros)
    SU, ML, MR = _mid_call(SU, MSGL, MSGR, Ws1, bs1, Wm1)
    MSGL, MSGR = seg2(ML, MR, SRC2, DST2, zeros)
    SU2 = _fuse_call(SU, MSGL, MSGR, Wself_u[2][None], bias_u[2][None, None], 0, HID)
    M2L, M2R = _fuse_msg_call(SU, MSGL, MSGR, Wmsg_i2u[2][None], NB_U)
    seg1 = _make_seg(1)
    MSG2L, MSG2R = seg1(M2L, M2R, SRC1, DST1, zeros)
    out = _fuse_call(SU2, MSG2L, MSG2R, Wmlp[None], bmlp[None, None], 0, OUTD)
    return out[:NU]


# speedup vs baseline: 2.2284x; 1.0006x over previous
import functools

import jax
import jax.numpy as jnp
from jax import lax
from jax.experimental import pallas as pl
from jax.experimental.pallas import tpu as pltpu
from jax.experimental.pallas import tpu_sc as plsc

f32 = jnp.float32
i32 = jnp.int32

NU = 10000
NI = 10000
NP = 10240
NTOT = 2 * NP
E = 160000
NCOLS = 4
HID = 256
HALF = 128
OUTD = 64

NS = 16
CHUNK = 128
CPP = 1280
EPAD = CPP * CHUNK
CPS = CPP // NS
ACC_ROWS = NP
TRASH = 10100
ZROWS = ACC_ROWS // NS
OROWS = ZROWS
NBUF = 2
WIN = 16
NWIN = CPS // WIN

RB = 640
NB_U = NP // RB


def _a0_body(x_ref, ws_ref, bs_ref, wm_ref, bm_ref, su_ref, ml_ref, mr_ref):
    x = x_ref[...]
    su_ref[...] = jnp.dot(x, ws_ref[0], preferred_element_type=f32) + bs_ref[0]
    m = jnp.dot(x, wm_ref[0], preferred_element_type=f32) + bm_ref[0]
    ml_ref[...] = m[:, :HALF]
    mr_ref[...] = m[:, HALF:]


def _mid_body(su_ref, ml_ref, mr_ref, ws_ref, bs_ref, wm_ref, su_o, ml_o, mr_o):
    h = su_ref[...] + jnp.concatenate([ml_ref[...], mr_ref[...]], axis=1)
    h = jnp.maximum(h, 0.0)
    su_o[...] = jnp.dot(h, ws_ref[0], preferred_element_type=f32) + bs_ref[0]
    m = jnp.dot(h, wm_ref[0], preferred_element_type=f32)
    ml_o[...] = m[:, :HALF]
    mr_o[...] = m[:, HALF:]


def _fuse_body(su_ref, ml_ref, mr_ref, w_ref, b_ref, o_ref):
    h = su_ref[...] + jnp.concatenate([ml_ref[...], mr_ref[...]], axis=1)
    h = jnp.maximum(h, 0.0)
    o_ref[...] = jnp.dot(h, w_ref[0], preferred_element_type=f32) + b_ref[0]


def _fuse_msg_body(su_ref, ml_ref, mr_ref, w_ref, ml_o, mr_o):
    h = su_ref[...] + jnp.concatenate([ml_ref[...], mr_ref[...]], axis=1)
    h = jnp.maximum(h, 0.0)
    m = jnp.dot(h, w_ref[0], preferred_element_type=f32)
    ml_o[...] = m[:, :HALF]
    mr_o[...] = m[:, HALF:]


def _rows(cols, off=0):
    return pl.BlockSpec((RB, cols), lambda i, o=off: (i + o, 0))


def _wspec(a, b):
    return pl.BlockSpec((1, a, b), lambda i: (i // NB_U, 0, 0))


def _bspec(b):
    return pl.BlockSpec((1, 1, b), lambda i: (i // NB_U, 0, 0))


def _wfix(a, b):
    return pl.BlockSpec((1, a, b), lambda i: (0, 0, 0))


def _bfix(b):
    return pl.BlockSpec((1, 1, b), lambda i: (0, 0, 0))


def _sds(r, c):
    return jax.ShapeDtypeStruct((r, c), f32)


def _a0_call(x, ws, bs, wm, bm):
    return pl.pallas_call(
        _a0_body,
        grid=(2 * NB_U,),
        in_specs=[_rows(NCOLS), _wspec(NCOLS, HID), _bspec(HID),
                  _wspec(NCOLS, HID), _bspec(HID)],
        out_specs=[_rows(HID), _rows(HALF), _rows(HALF)],
        out_shape=[_sds(NTOT, HID), _sds(NTOT, HALF), _sds(NTOT, HALF)],
    )(x, ws, bs, wm, bm)


def _mid_call(su, ml, mr, ws, bs, wm):
    return pl.pallas_call(
        _mid_body,
        grid=(2 * NB_U,),
        in_specs=[_rows(HID), _rows(HALF), _rows(HALF),
                  _wspec(HID, HID), _bspec(HID), _wspec(HID, HID)],
        out_specs=[_rows(HID), _rows(HALF), _rows(HALF)],
        out_shape=[_sds(NTOT, HID), _sds(NTOT, HALF), _sds(NTOT, HALF)],
    )(su, ml, mr, ws, bs, wm)


def _fuse_call(su, ml, mr, w, b, off, outc):
    return pl.pallas_call(
        _fuse_body,
        grid=(NB_U,),
        in_specs=[_rows(HID, off), _rows(HALF, off), _rows(HALF, off),
                  _wfix(HID, outc), _bfix(outc)],
        out_specs=_rows(outc),
        out_shape=_sds(NP, outc),
    )(su, ml, mr, w, b)


def _fuse_msg_call(su, ml, mr, w, off):
    return pl.pallas_call(
        _fuse_msg_body,
        grid=(NB_U,),
        in_specs=[_rows(HID, off), _rows(HALF, off), _rows(HALF, off),
                  _wfix(HID, HID)],
        out_specs=[_rows(HALF), _rows(HALF)],
        out_shape=[_sds(NP, HALF), _sds(NP, HALF)],
    )(su, ml, mr, w)


def _make_seg(nphase):
    out_rows = nphase * NP
    mesh = plsc.VectorSubcoreMesh(core_axis_name="c", subcore_axis_name="s")

    @functools.partial(
        pl.kernel,
        out_type=(jax.ShapeDtypeStruct((out_rows, HALF), f32),
                  jax.ShapeDtypeStruct((out_rows, HALF), f32)),
        mesh=mesh,
        scratch_types=(
            pltpu.VMEM((2, WIN, CHUNK), i32),
            pltpu.VMEM((2, WIN, CHUNK), i32),
            pltpu.VMEM((NBUF, CHUNK, HALF), f32),
            pltpu.VMEM_SHARED((ACC_ROWS, HALF), f32),
            pltpu.SemaphoreType.DMA((NBUF,)),
            pltpu.SemaphoreType.DMA((NBUF,)),
        ),
    )
    def seg(ml, mr, srcs, dsts, zeros, outl, outr, srcv, dstv, bufs, accum,
            gsem, ssem):
        c = lax.axis_index("c")
        s = lax.axis_index("s")

        def run(tab, out):
            for p in range(nphase):
                base = p * CPP + s * CPS
                pltpu.sync_copy(zeros, accum.at[pl.ds(s * ZROWS, ZROWS), :])
                pltpu.sync_copy(srcs.at[pl.ds(base, WIN), :], srcv.at[0])
                pltpu.sync_copy(dsts.at[pl.ds(base, WIN), :], dstv.at[0])
                pltpu.sync_copy(srcs.at[pl.ds(base + WIN, WIN), :], srcv.at[1])
                pltpu.sync_copy(dsts.at[pl.ds(base + WIN, WIN), :], dstv.at[1])
                plsc.subcore_barrier()

                def gather_chunk(sl, row, b):
                    for h in range(2):
                        hs = pl.ds(h * 64, 64)
                        pltpu.async_copy(tab.at[srcv.at[sl, row, hs]],
                                         bufs.at[b, hs, :], gsem.at[b])

                for b in range(NBUF):
                    gather_chunk(0, b, b)

                def group(i0, _):
                    g0 = i0 * NBUF
                    w = g0 // WIN

                    @pl.when((lax.rem(g0, WIN) == 0) & (w >= 1)
                             & (w + 1 < NWIN))
                    def _():
                        sl = lax.rem(w + 1, 2)
                        pltpu.sync_copy(
                            srcs.at[pl.ds(base + (w + 1) * WIN, WIN), :],
                            srcv.at[sl])
                        pltpu.sync_copy(
                            dsts.at[pl.ds(base + (w + 1) * WIN, WIN), :],
                            dstv.at[sl])

                    for b in range(NBUF):
                        g = g0 + b
                        sl = lax.rem(g // WIN, 2)
                        row = lax.rem(g, WIN)
                        pltpu.make_async_copy(tab.at[srcv.at[sl, row]],
                                              bufs.at[b], gsem.at[b]).wait()
                        pltpu.async_copy(bufs.at[b],
                                         accum.at[dstv.at[sl, row]],
                                         ssem.at[b], add=True)
                    for b in range(NBUF):
                        g = g0 + b
                        g2 = g + NBUF
                        sl = lax.rem(g // WIN, 2)
                        row = lax.rem(g, WIN)
                        sl2 = lax.rem(g2 // WIN, 2)
                        row2 = lax.rem(g2, WIN)
                        pltpu.make_async_copy(bufs.at[b],
                                              accum.at[dstv.at[sl, row]],
                                              ssem.at[b]).wait()

                        @pl.when(g2 < CPS)
                        def _():
                            gather_chunk(sl2, row2, b)
                    return 0

                lax.fori_loop(0, CPS // NBUF, group, 0)
                plsc.subcore_barrier()
                pltpu.sync_copy(accum.at[pl.ds(s * OROWS, OROWS), :],
                                out.at[pl.ds(p * NP + s * OROWS, OROWS), :])
                plsc.subcore_barrier()

        @pl.when(c == 0)
        def _():
            run(ml, outl)

        @pl.when(c == 1)
        def _():
            run(mr, outr)

    return seg


def kernel(x_user, x_item, edge_u2i, edge_i2u, Wemb_u, bemb_u, Wemb_i, bemb_i,
           Wself_u, Wself_i, bias_u, bias_i, Wmsg_u2i, Wmsg_i2u, Wmlp, bmlp):
    eye = jnp.eye(NCOLS, dtype=f32)
    Wbig_u = (eye[:, :, None] * Wemb_u[None]).reshape(NCOLS, HID)
    Wbig_i = (eye[:, :, None] * Wemb_i[None]).reshape(NCOLS, HID)
    bflat_u = bemb_u.reshape(HID)
    bflat_i = bemb_i.reshape(HID)
    WsF = jnp.stack([Wbig_u @ Wself_u[0], Wbig_i @ Wself_i[0]])
    bsF = jnp.stack([bflat_u @ Wself_u[0] + bias_u[0],
                     bflat_i @ Wself_i[0] + bias_i[0]])[:, None, :]
    WmF = jnp.stack([Wbig_u @ Wmsg_u2i[0], Wbig_i @ Wmsg_i2u[0]])
    bmF = jnp.stack([bflat_u @ Wmsg_u2i[0], bflat_i @ Wmsg_i2u[0]])[:, None, :]
    Ws1 = jnp.stack([Wself_u[1], Wself_i[1]])
    bs1 = jnp.stack([bias_u[1], bias_i[1]])[:, None, :]
    Wm1 = jnp.stack([Wmsg_u2i[1], Wmsg_i2u[1]])

    src_u2i = edge_u2i[0].astype(i32)
    dst_u2i = edge_u2i[1].astype(i32)
    src_i2u = edge_i2u[0].astype(i32)
    dst_i2u = edge_i2u[1].astype(i32)
    pad_s = jnp.zeros((EPAD - E,), i32)
    pad_d = jnp.full((EPAD - E,), TRASH, i32)
    SRC2 = jnp.concatenate([src_i2u + NP, pad_s, src_u2i, pad_s]).reshape(2 * CPP, CHUNK)
    DST2 = jnp.concatenate([dst_i2u, pad_d, dst_u2i, pad_d]).reshape(2 * CPP, CHUNK)
    SRC1 = jnp.concatenate([src_i2u, pad_s]).reshape(CPP, CHUNK)
    DST1 = jnp.concatenate([dst_i2u, pad_d]).reshape(CPP, CHUNK)
    zeros = jnp.zeros((ZROWS, HALF), f32)

    X = jnp.concatenate([jnp.pad(x_user, ((0, NP - NU), (0, 0))),
                         jnp.pad(x_item, ((0, NP - NI), (0, 0)))], axis=0)

    SU, ML, MR = _a0_call(X, WsF, bsF, WmF, bmF)
    seg2 = _make_seg(2)
    MSGL, MSGR = seg2(ML, MR, SRC2, DST2, zeros)
    SU, ML, MR = _mid_call(SU, MSGL, MSGR, Ws1, bs1, Wm1)
    MSGL, MSGR = seg2(ML, MR, SRC2, DST2, zeros)
    SU2 = _fuse_call(SU, MSGL, MSGR, Wself_u[2][None], bias_u[2][None, None], 0, HID)
    M2L, M2R = _fuse_msg_call(SU, MSGL, MSGR, Wmsg_i2u[2][None], NB_U)
    seg1 = _make_seg(1)
    MSG2L, MSG2R = seg1(M2L, M2R, SRC1, DST1, zeros)
    out = _fuse_call(SU2, MSG2L, MSG2R, Wmlp[None], bmlp[None, None], 0, OUTD)
    return out[:NU]


# speedup vs baseline: 2.4764x; 1.1113x over previous
import functools

import jax
import jax.numpy as jnp
from jax import lax
from jax.experimental import pallas as pl
from jax.experimental.pallas import tpu as pltpu
from jax.experimental.pallas import tpu_sc as plsc

f32 = jnp.float32
i32 = jnp.int32

NU = 10000
NI = 10000
NP = 10240
NTOT = 2 * NP
E = 160000
NCOLS = 4
HID = 256
HALF = 128
OUTD = 64

NS = 16
CHUNK = 128
CPP = 1280
EPAD = CPP * CHUNK
CPS = CPP // NS
ACC_ROWS = NP
TRASH = 10100
ZROWS = ACC_ROWS // NS
OROWS = ZROWS
NBUF = 2
WIN = 16
NWIN = CPS // WIN

RB = 640
NB_U = NP // RB


def _a0_body(x_ref, ws_ref, bs_ref, wm_ref, bm_ref, su_ref, ml_ref, mr_ref):
    x = x_ref[...]
    su_ref[...] = jnp.dot(x, ws_ref[0], preferred_element_type=f32) + bs_ref[0]
    m = jnp.dot(x, wm_ref[0], preferred_element_type=f32) + bm_ref[0]
    ml_ref[...] = m[:, :HALF]
    mr_ref[...] = m[:, HALF:]


def _mid_body(su_ref, ml_ref, mr_ref, ws_ref, bs_ref, wm_ref, su_o, ml_o, mr_o):
    h = su_ref[...] + jnp.concatenate([ml_ref[...], mr_ref[...]], axis=1)
    h = jnp.maximum(h, 0.0)
    su_o[...] = jnp.dot(h, ws_ref[0], preferred_element_type=f32) + bs_ref[0]
    m = jnp.dot(h, wm_ref[0], preferred_element_type=f32)
    ml_o[...] = m[:, :HALF]
    mr_o[...] = m[:, HALF:]


def _fuse_body(su_ref, ml_ref, mr_ref, w_ref, b_ref, o_ref):
    h = su_ref[...] + jnp.concatenate([ml_ref[...], mr_ref[...]], axis=1)
    h = jnp.maximum(h, 0.0)
    o_ref[...] = jnp.dot(h, w_ref[0], preferred_element_type=f32) + b_ref[0]


def _fuse_msg_body(su_ref, ml_ref, mr_ref, w_ref, ml_o, mr_o):
    h = su_ref[...] + jnp.concatenate([ml_ref[...], mr_ref[...]], axis=1)
    h = jnp.maximum(h, 0.0)
    m = jnp.dot(h, w_ref[0], preferred_element_type=f32)
    ml_o[...] = m[:, :HALF]
    mr_o[...] = m[:, HALF:]


def _rows(cols, off=0):
    return pl.BlockSpec((RB, cols), lambda i, o=off: (i + o, 0))


def _wspec(a, b):
    return pl.BlockSpec((1, a, b), lambda i: (i // NB_U, 0, 0))


def _bspec(b):
    return pl.BlockSpec((1, 1, b), lambda i: (i // NB_U, 0, 0))


def _wfix(a, b):
    return pl.BlockSpec((1, a, b), lambda i: (0, 0, 0))


def _bfix(b):
    return pl.BlockSpec((1, 1, b), lambda i: (0, 0, 0))


def _sds(r, c):
    return jax.ShapeDtypeStruct((r, c), f32)


def _a0_call(x, ws, bs, wm, bm):
    return pl.pallas_call(
        _a0_body,
        grid=(2 * NB_U,),
        in_specs=[_rows(NCOLS), _wspec(NCOLS, HID), _bspec(HID),
                  _wspec(NCOLS, HID), _bspec(HID)],
        out_specs=[_rows(HID), _rows(HALF), _rows(HALF)],
        out_shape=[_sds(NTOT, HID), _sds(NTOT, HALF), _sds(NTOT, HALF)],
    )(x, ws, bs, wm, bm)


def _mid_call(su, ml, mr, ws, bs, wm):
    return pl.pallas_call(
        _mid_body,
        grid=(2 * NB_U,),
        in_specs=[_rows(HID), _rows(HALF), _rows(HALF),
                  _wspec(HID, HID), _bspec(HID), _wspec(HID, HID)],
        out_specs=[_rows(HID), _rows(HALF), _rows(HALF)],
        out_shape=[_sds(NTOT, HID), _sds(NTOT, HALF), _sds(NTOT, HALF)],
    )(su, ml, mr, ws, bs, wm)


def _fuse_call(su, ml, mr, w, b, off, outc):
    return pl.pallas_call(
        _fuse_body,
        grid=(NB_U,),
        in_specs=[_rows(HID, off), _rows(HALF, off), _rows(HALF, off),
                  _wfix(HID, outc), _bfix(outc)],
        out_specs=_rows(outc),
        out_shape=_sds(NP, outc),
    )(su, ml, mr, w, b)


def _fuse_msg_call(su, ml, mr, w, off):
    return pl.pallas_call(
        _fuse_msg_body,
        grid=(NB_U,),
        in_specs=[_rows(HID, off), _rows(HALF, off), _rows(HALF, off),
                  _wfix(HID, HID)],
        out_specs=[_rows(HALF), _rows(HALF)],
        out_shape=[_sds(NP, HALF), _sds(NP, HALF)],
    )(su, ml, mr, w)


def _make_seg(nphase):
    out_rows = nphase * NP
    mesh = plsc.VectorSubcoreMesh(core_axis_name="c", subcore_axis_name="s")

    @functools.partial(
        pl.kernel,
        out_type=(jax.ShapeDtypeStruct((out_rows, HALF), f32),
                  jax.ShapeDtypeStruct((out_rows, HALF), f32)),
        mesh=mesh,
        scratch_types=(
            pltpu.VMEM((2, WIN, CHUNK), i32),
            pltpu.VMEM((2, WIN, CHUNK), i32),
            pltpu.VMEM((NBUF, CHUNK, HALF), f32),
            pltpu.VMEM_SHARED((ACC_ROWS, HALF), f32),
            pltpu.SemaphoreType.DMA((NBUF,)),
            pltpu.SemaphoreType.DMA((NBUF,)),
        ),
    )
    def seg(ml, mr, srcs, dsts, zeros, outl, outr, srcv, dstv, bufs, accum,
            gsem, ssem):
        c = lax.axis_index("c")
        s = lax.axis_index("s")

        def run(tab, out):
            for p in range(nphase):
                base = p * CPP + s * CPS
                pltpu.sync_copy(zeros, accum.at[pl.ds(s * ZROWS, ZROWS), :])
                pltpu.sync_copy(srcs.at[pl.ds(base, WIN), :], srcv.at[0])
                pltpu.sync_copy(dsts.at[pl.ds(base, WIN), :], dstv.at[0])
                pltpu.sync_copy(srcs.at[pl.ds(base + WIN, WIN), :], srcv.at[1])
                pltpu.sync_copy(dsts.at[pl.ds(base + WIN, WIN), :], dstv.at[1])
                plsc.subcore_barrier()

                def gather_chunk(sl, row, b):
                    for h in range(2):
                        hs = pl.ds(h * 64, 64)
                        pltpu.async_copy(tab.at[srcv.at[sl, row, hs]],
                                         bufs.at[b, hs, :], gsem.at[b])

                for b in range(NBUF):
                    gather_chunk(0, b, b)

                def group(i0, _):
                    g0 = i0 * NBUF
                    w = g0 // WIN

                    @pl.when((lax.rem(g0, WIN) == 0) & (w >= 1)
                             & (w + 1 < NWIN))
                    def _():
                        sl = lax.rem(w + 1, 2)
                        pltpu.sync_copy(
                            srcs.at[pl.ds(base + (w + 1) * WIN, WIN), :],
                            srcv.at[sl])
                        pltpu.sync_copy(
                            dsts.at[pl.ds(base + (w + 1) * WIN, WIN), :],
                            dstv.at[sl])

                    for b in range(NBUF):
                        g = g0 + b
                        sl = lax.rem(g // WIN, 2)
                        row = lax.rem(g, WIN)
                        pltpu.make_async_copy(tab.at[srcv.at[sl, row]],
                                              bufs.at[b], gsem.at[b]).wait()
                        pltpu.async_copy(bufs.at[b],
                                         accum.at[dstv.at[sl, row]],
                                         ssem.at[b], add=True)
                    for b in range(NBUF):
                        g = g0 + b
                        g2 = g + NBUF
                        sl = lax.rem(g // WIN, 2)
                        row = lax.rem(g, WIN)
                        sl2 = lax.rem(g2 // WIN, 2)
                        row2 = lax.rem(g2, WIN)
                        pltpu.make_async_copy(bufs.at[b],
                                              accum.at[dstv.at[sl, row]],
                                              ssem.at[b]).wait()

                        @pl.when(g2 < CPS)
                        def _():
                            gather_chunk(sl2, row2, b)
                    return 0

                lax.fori_loop(0, CPS // NBUF, group, 0)
                plsc.subcore_barrier()
                pltpu.sync_copy(accum.at[pl.ds(s * OROWS, OROWS), :],
                                out.at[pl.ds(p * NP + s * OROWS, OROWS), :])
                plsc.subcore_barrier()

        @pl.when(c == 0)
        def _():
            run(ml, outl)

        @pl.when(c == 1)
        def _():
            run(mr, outr)

    return seg


def kernel(x_user, x_item, edge_u2i, edge_i2u, Wemb_u, bemb_u, Wemb_i, bemb_i,
           Wself_u, Wself_i, bias_u, bias_i, Wmsg_u2i, Wmsg_i2u, Wmlp, bmlp):
    eye = jnp.eye(NCOLS, dtype=f32)
    Wbig_u = (eye[:, :, None] * Wemb_u[None]).reshape(NCOLS, HID)
    Wbig_i = (eye[:, :, None] * Wemb_i[None]).reshape(NCOLS, HID)
    bflat_u = bemb_u.reshape(HID)
    bflat_i = bemb_i.reshape(HID)
    WsF = jnp.stack([Wbig_u @ Wself_u[0], Wbig_i @ Wself_i[0]])
    bsF = jnp.stack([bflat_u @ Wself_u[0] + bias_u[0],
                     bflat_i @ Wself_i[0] + bias_i[0]])[:, None, :]
    WmF = jnp.stack([Wbig_u @ Wmsg_u2i[0], Wbig_i @ Wmsg_i2u[0]])
    bmF = jnp.stack([bflat_u @ Wmsg_u2i[0], bflat_i @ Wmsg_i2u[0]])[:, None, :]
    Ws1 = jnp.stack([Wself_u[1], Wself_i[1]])
    bs1 = jnp.stack([bias_u[1], bias_i[1]])[:, None, :]
    Wm1 = jnp.stack([Wmsg_u2i[1], Wmsg_i2u[1]])

    src_u2i = edge_u2i[0].astype(i32)
    dst_u2i = edge_u2i[1].astype(i32)
    src_i2u = edge_i2u[0].astype(i32)
    dst_i2u = edge_i2u[1].astype(i32)
    ord_i2u = jnp.argsort(src_i2u)
    src_i2u, dst_i2u = src_i2u[ord_i2u], dst_i2u[ord_i2u]
    ord_u2i = jnp.argsort(src_u2i)
    src_u2i, dst_u2i = src_u2i[ord_u2i], dst_u2i[ord_u2i]
    npad = EPAD - E
    pad_s = (jnp.arange(npad, dtype=i32) * 13) % NU
    pad_d = NU + 64 + (jnp.arange(npad, dtype=i32) % 128)
    SRC2 = jnp.concatenate([src_i2u + NP, pad_s, src_u2i, pad_s]).reshape(2 * CPP, CHUNK)
    DST2 = jnp.concatenate([dst_i2u, pad_d, dst_u2i, pad_d]).reshape(2 * CPP, CHUNK)
    SRC1 = jnp.concatenate([src_i2u, pad_s]).reshape(CPP, CHUNK)
    DST1 = jnp.concatenate([dst_i2u, pad_d]).reshape(CPP, CHUNK)
    zeros = jnp.zeros((ZROWS, HALF), f32)

    X = jnp.concatenate([jnp.pad(x_user, ((0, NP - NU), (0, 0))),
                         jnp.pad(x_item, ((0, NP - NI), (0, 0)))], axis=0)

    SU, ML, MR = _a0_call(X, WsF, bsF, WmF, bmF)
    seg2 = _make_seg(2)
    MSGL, MSGR = seg2(ML, MR, SRC2, DST2, zeros)
    SU, ML, MR = _mid_call(SU, MSGL, MSGR, Ws1, bs1, Wm1)
    MSGL, MSGR = seg2(ML, MR, SRC2, DST2, zeros)
    SU2 = _fuse_call(SU, MSGL, MSGR, Wself_u[2][None], bias_u[2][None, None], 0, HID)
    M2L, M2R = _fuse_msg_call(SU, MSGL, MSGR, Wmsg_i2u[2][None], NB_U)
    seg1 = _make_seg(1)
    MSG2L, MSG2R = seg1(M2L, M2R, SRC1, DST1, zeros)
    out = _fuse_call(SU2, MSG2L, MSG2R, Wmlp[None], bmlp[None, None], 0, OUTD)
    return out[:NU]


# speedup vs baseline: 2.5859x; 1.0442x over previous
import functools

import jax
import jax.numpy as jnp
from jax import lax
from jax.experimental import pallas as pl
from jax.experimental.pallas import tpu as pltpu
from jax.experimental.pallas import tpu_sc as plsc

f32 = jnp.float32
i32 = jnp.int32

NU = 10000
NI = 10000
NP = 10240
NTOT = 2 * NP
E = 160000
NCOLS = 4
HID = 256
HALF = 128
OUTD = 64

NS = 16
CHUNK = 128
CPP = 1280
EPAD = CPP * CHUNK
CPS = CPP // NS
ACC_ROWS = NP
TRASH = 10100
ZROWS = ACC_ROWS // NS
OROWS = ZROWS
NBUF = 2
WIN = 16
NWIN = CPS // WIN

RB = 640
NB_U = NP // RB


def _a0_body(x_ref, ws_ref, bs_ref, wm_ref, bm_ref, su_ref, ml_ref, mr_ref):
    x = x_ref[...]
    su_ref[...] = jnp.dot(x, ws_ref[0], preferred_element_type=f32) + bs_ref[0]
    m = jnp.dot(x, wm_ref[0], preferred_element_type=f32) + bm_ref[0]
    ml_ref[...] = m[:, :HALF]
    mr_ref[...] = m[:, HALF:]


def _mid_body(su_ref, ml_ref, mr_ref, ws_ref, bs_ref, wm_ref, su_o, ml_o, mr_o):
    h = su_ref[...] + jnp.concatenate([ml_ref[...], mr_ref[...]], axis=1)
    h = jnp.maximum(h, 0.0)
    su_o[...] = jnp.dot(h, ws_ref[0], preferred_element_type=f32) + bs_ref[0]
    m = jnp.dot(h, wm_ref[0], preferred_element_type=f32)
    ml_o[...] = m[:, :HALF]
    mr_o[...] = m[:, HALF:]


def _fuse_body(su_ref, ml_ref, mr_ref, w_ref, b_ref, o_ref):
    h = su_ref[...] + jnp.concatenate([ml_ref[...], mr_ref[...]], axis=1)
    h = jnp.maximum(h, 0.0)
    o_ref[...] = jnp.dot(h, w_ref[0], preferred_element_type=f32) + b_ref[0]


def _fuse_msg_body(su_ref, ml_ref, mr_ref, w_ref, ml_o, mr_o):
    h = su_ref[...] + jnp.concatenate([ml_ref[...], mr_ref[...]], axis=1)
    h = jnp.maximum(h, 0.0)
    m = jnp.dot(h, w_ref[0], preferred_element_type=f32)
    ml_o[...] = m[:, :HALF]
    mr_o[...] = m[:, HALF:]


def _rows(cols, off=0):
    return pl.BlockSpec((RB, cols), lambda i, o=off: (i + o, 0))


def _wspec(a, b):
    return pl.BlockSpec((1, a, b), lambda i: (i // NB_U, 0, 0))


def _bspec(b):
    return pl.BlockSpec((1, 1, b), lambda i: (i // NB_U, 0, 0))


def _wfix(a, b):
    return pl.BlockSpec((1, a, b), lambda i: (0, 0, 0))


def _bfix(b):
    return pl.BlockSpec((1, 1, b), lambda i: (0, 0, 0))


def _sds(r, c):
    return jax.ShapeDtypeStruct((r, c), f32)


def _a0_call(x, ws, bs, wm, bm):
    return pl.pallas_call(
        _a0_body,
        grid=(2 * NB_U,),
        in_specs=[_rows(NCOLS), _wspec(NCOLS, HID), _bspec(HID),
                  _wspec(NCOLS, HID), _bspec(HID)],
        out_specs=[_rows(HID), _rows(HALF), _rows(HALF)],
        out_shape=[_sds(NTOT, HID), _sds(NTOT, HALF), _sds(NTOT, HALF)],
    )(x, ws, bs, wm, bm)


def _mid_call(su, ml, mr, ws, bs, wm):
    return pl.pallas_call(
        _mid_body,
        grid=(2 * NB_U,),
        in_specs=[_rows(HID), _rows(HALF), _rows(HALF),
                  _wspec(HID, HID), _bspec(HID), _wspec(HID, HID)],
        out_specs=[_rows(HID), _rows(HALF), _rows(HALF)],
        out_shape=[_sds(NTOT, HID), _sds(NTOT, HALF), _sds(NTOT, HALF)],
    )(su, ml, mr, ws, bs, wm)


def _fuse_call(su, ml, mr, w, b, off, outc):
    return pl.pallas_call(
        _fuse_body,
        grid=(NB_U,),
        in_specs=[_rows(HID, off), _rows(HALF, off), _rows(HALF, off),
                  _wfix(HID, outc), _bfix(outc)],
        out_specs=_rows(outc),
        out_shape=_sds(NP, outc),
    )(su, ml, mr, w, b)


def _fuse_msg_call(su, ml, mr, w, off):
    return pl.pallas_call(
        _fuse_msg_body,
        grid=(NB_U,),
        in_specs=[_rows(HID, off), _rows(HALF, off), _rows(HALF, off),
                  _wfix(HID, HID)],
        out_specs=[_rows(HALF), _rows(HALF)],
        out_shape=[_sds(NP, HALF), _sds(NP, HALF)],
    )(su, ml, mr, w)


def _make_seg(nphase):
    out_rows = nphase * NP
    mesh = plsc.VectorSubcoreMesh(core_axis_name="c", subcore_axis_name="s")

    @functools.partial(
        pl.kernel,
        out_type=(jax.ShapeDtypeStruct((out_rows, HALF), f32),
                  jax.ShapeDtypeStruct((out_rows, HALF), f32)),
        mesh=mesh,
        scratch_types=(
            pltpu.VMEM((2, WIN, CHUNK), i32),
            pltpu.VMEM((2, WIN, CHUNK), i32),
            pltpu.VMEM((NBUF, CHUNK, HALF), f32),
            pltpu.VMEM_SHARED((ACC_ROWS, HALF), f32),
            pltpu.SemaphoreType.DMA((NBUF,)),
            pltpu.SemaphoreType.DMA((NBUF,)),
        ),
    )
    def seg(ml, mr, srcs, dsts, zeros, outl, outr, srcv, dstv, bufs, accum,
            gsem, ssem):
        c = lax.axis_index("c")
        s = lax.axis_index("s")

        def run(tab, out):
            for p in range(nphase):
                base = p * CPP + s * CPS
                pltpu.sync_copy(zeros, accum.at[pl.ds(s * ZROWS, ZROWS), :])
                pltpu.sync_copy(srcs.at[pl.ds(base, WIN), :], srcv.at[0])
                pltpu.sync_copy(dsts.at[pl.ds(base, WIN), :], dstv.at[0])
                pltpu.sync_copy(srcs.at[pl.ds(base + WIN, WIN), :], srcv.at[1])
                pltpu.sync_copy(dsts.at[pl.ds(base + WIN, WIN), :], dstv.at[1])
                plsc.subcore_barrier()

                def gather_chunk(sl, row, b):
                    for h in range(2):
                        hs = pl.ds(h * 64, 64)
                        pltpu.async_copy(tab.at[srcv.at[sl, row, hs]],
                                         bufs.at[b, hs, :], gsem.at[b])

                for b in range(NBUF):
                    gather_chunk(0, b, b)

                def group(i0, _):
                    g0 = i0 * NBUF
                    w = g0 // WIN

                    @pl.when((lax.rem(g0, WIN) == 0) & (w >= 1)
                             & (w + 1 < NWIN))
                    def _():
                        sl = lax.rem(w + 1, 2)
                        pltpu.sync_copy(
                            srcs.at[pl.ds(base + (w + 1) * WIN, WIN), :],
                            srcv.at[sl])
                        pltpu.sync_copy(
                            dsts.at[pl.ds(base + (w + 1) * WIN, WIN), :],
                            dstv.at[sl])

                    for b in range(NBUF):
                        g = g0 + b
                        sl = lax.rem(g // WIN, 2)
                        row = lax.rem(g, WIN)
                        pltpu.make_async_copy(tab.at[srcv.at[sl, row]],
                                              bufs.at[b], gsem.at[b]).wait()
                        pltpu.async_copy(bufs.at[b],
                                         accum.at[dstv.at[sl, row]],
                                         ssem.at[b], add=True)
                    for b in range(NBUF):
                        g = g0 + b
                        g2 = g + NBUF
                        sl = lax.rem(g // WIN, 2)
                        row = lax.rem(g, WIN)
                        sl2 = lax.rem(g2 // WIN, 2)
                        row2 = lax.rem(g2, WIN)
                        pltpu.make_async_copy(bufs.at[b],
                                              accum.at[dstv.at[sl, row]],
                                              ssem.at[b]).wait()

                        @pl.when(g2 < CPS)
                        def _():
                            gather_chunk(sl2, row2, b)
                    return 0

                lax.fori_loop(0, CPS // NBUF, group, 0)
                plsc.subcore_barrier()
                pltpu.sync_copy(accum.at[pl.ds(s * OROWS, OROWS), :],
                                out.at[pl.ds(p * NP + s * OROWS, OROWS), :])
                plsc.subcore_barrier()

        @pl.when(c == 0)
        def _():
            run(ml, outl)

        @pl.when(c == 1)
        def _():
            run(mr, outr)

    return seg


def kernel(x_user, x_item, edge_u2i, edge_i2u, Wemb_u, bemb_u, Wemb_i, bemb_i,
           Wself_u, Wself_i, bias_u, bias_i, Wmsg_u2i, Wmsg_i2u, Wmlp, bmlp):
    eye = jnp.eye(NCOLS, dtype=f32)
    Wbig_u = (eye[:, :, None] * Wemb_u[None]).reshape(NCOLS, HID)
    Wbig_i = (eye[:, :, None] * Wemb_i[None]).reshape(NCOLS, HID)
    bflat_u = bemb_u.reshape(HID)
    bflat_i = bemb_i.reshape(HID)
    WsF = jnp.stack([Wbig_u @ Wself_u[0], Wbig_i @ Wself_i[0]])
    bsF = jnp.stack([bflat_u @ Wself_u[0] + bias_u[0],
                     bflat_i @ Wself_i[0] + bias_i[0]])[:, None, :]
    WmF = jnp.stack([Wbig_u @ Wmsg_u2i[0], Wbig_i @ Wmsg_i2u[0]])
    bmF = jnp.stack([bflat_u @ Wmsg_u2i[0], bflat_i @ Wmsg_i2u[0]])[:, None, :]
    Ws1 = jnp.stack([Wself_u[1], Wself_i[1]])
    bs1 = jnp.stack([bias_u[1], bias_i[1]])[:, None, :]
    Wm1 = jnp.stack([Wmsg_u2i[1], Wmsg_i2u[1]])

    src_u2i = edge_u2i[0].astype(i32)
    dst_u2i = edge_u2i[1].astype(i32)
    src_i2u = edge_i2u[0].astype(i32)
    dst_i2u = edge_i2u[1].astype(i32)
    def _sort_by_src(srca, dsta):
        key = jnp.sort(srca * 16384 + dsta)
        return key >> 14, key & 16383
    src_i2u, dst_i2u = _sort_by_src(src_i2u, dst_i2u)
    src_u2i, dst_u2i = _sort_by_src(src_u2i, dst_u2i)
    npad = EPAD - E
    pad_s = (jnp.arange(npad, dtype=i32) * 13) % NU
    pad_d = NU + 64 + (jnp.arange(npad, dtype=i32) % 128)
    SRC2 = jnp.concatenate([src_i2u + NP, pad_s, src_u2i, pad_s]).reshape(2 * CPP, CHUNK)
    DST2 = jnp.concatenate([dst_i2u, pad_d, dst_u2i, pad_d]).reshape(2 * CPP, CHUNK)
    SRC1 = jnp.concatenate([src_i2u, pad_s]).reshape(CPP, CHUNK)
    DST1 = jnp.concatenate([dst_i2u, pad_d]).reshape(CPP, CHUNK)
    zeros = jnp.zeros((ZROWS, HALF), f32)

    X = jnp.concatenate([jnp.pad(x_user, ((0, NP - NU), (0, 0))),
                         jnp.pad(x_item, ((0, NP - NI), (0, 0)))], axis=0)

    SU, ML, MR = _a0_call(X, WsF, bsF, WmF, bmF)
    seg2 = _make_seg(2)
    MSGL, MSGR = seg2(ML, MR, SRC2, DST2, zeros)
    SU, ML, MR = _mid_call(SU, MSGL, MSGR, Ws1, bs1, Wm1)
    MSGL, MSGR = seg2(ML, MR, SRC2, DST2, zeros)
    SU2 = _fuse_call(SU, MSGL, MSGR, Wself_u[2][None], bias_u[2][None, None], 0, HID)
    M2L, M2R = _fuse_msg_call(SU, MSGL, MSGR, Wmsg_i2u[2][None], NB_U)
    seg1 = _make_seg(1)
    MSG2L, MSG2R = seg1(M2L, M2R, SRC1, DST1, zeros)
    out = _fuse_call(SU2, MSG2L, MSG2R, Wmlp[None], bmlp[None, None], 0, OUTD)
    return out[:NU]


# speedup vs baseline: 3.1394x; 1.2141x over previous
import functools

import jax
import jax.numpy as jnp
from jax import lax
from jax.experimental import pallas as pl
from jax.experimental.pallas import tpu as pltpu
from jax.experimental.pallas import tpu_sc as plsc

f32 = jnp.float32
i32 = jnp.int32

NU = 10000
NI = 10000
NP = 10240
NTOT = 2 * NP
E = 160000
NCOLS = 4
HID = 256
HALF = 128
OUTD = 64

NS = 16
CHUNK = 128
CPP = 1280
EPAD = CPP * CHUNK
CPS = CPP // NS
ACC_ROWS = NP
TRASH = 10100
ZROWS = ACC_ROWS // NS
OROWS = ZROWS
NBUF = 2
WIN = 16
NWIN = CPS // WIN

RB = 640
NB_U = NP // RB


def _a0_body(x_ref, ws_ref, bs_ref, wm_ref, bm_ref, su_ref, ml_ref, mr_ref):
    x = x_ref[...]
    su_ref[...] = jnp.dot(x, ws_ref[0], preferred_element_type=f32) + bs_ref[0]
    m = jnp.dot(x, wm_ref[0], preferred_element_type=f32) + bm_ref[0]
    ml_ref[...] = m[:, :HALF]
    mr_ref[...] = m[:, HALF:]


def _mid_body(su_ref, ml_ref, mr_ref, ws_ref, bs_ref, wm_ref, su_o, ml_o, mr_o):
    h = su_ref[...] + jnp.concatenate([ml_ref[...], mr_ref[...]], axis=1)
    h = jnp.maximum(h, 0.0)
    su_o[...] = jnp.dot(h, ws_ref[0], preferred_element_type=f32) + bs_ref[0]
    m = jnp.dot(h, wm_ref[0], preferred_element_type=f32)
    ml_o[...] = m[:, :HALF]
    mr_o[...] = m[:, HALF:]


def _fuse_body(su_ref, ml_ref, mr_ref, w_ref, b_ref, o_ref):
    h = su_ref[...] + jnp.concatenate([ml_ref[...], mr_ref[...]], axis=1)
    h = jnp.maximum(h, 0.0)
    o_ref[...] = jnp.dot(h, w_ref[0], preferred_element_type=f32) + b_ref[0]


def _fuse_msg_body(su_ref, ml_ref, mr_ref, w_ref, ml_o, mr_o):
    h = su_ref[...] + jnp.concatenate([ml_ref[...], mr_ref[...]], axis=1)
    h = jnp.maximum(h, 0.0)
    m = jnp.dot(h, w_ref[0], preferred_element_type=f32)
    ml_o[...] = m[:, :HALF]
    mr_o[...] = m[:, HALF:]


def _rows(cols, off=0):
    return pl.BlockSpec((RB, cols), lambda i, o=off: (i + o, 0))


def _wspec(a, b):
    return pl.BlockSpec((1, a, b), lambda i: (i // NB_U, 0, 0))


def _bspec(b):
    return pl.BlockSpec((1, 1, b), lambda i: (i // NB_U, 0, 0))


def _wfix(a, b):
    return pl.BlockSpec((1, a, b), lambda i: (0, 0, 0))


def _bfix(b):
    return pl.BlockSpec((1, 1, b), lambda i: (0, 0, 0))


def _sds(r, c):
    return jax.ShapeDtypeStruct((r, c), f32)


def _a0_call(x, ws, bs, wm, bm):
    return pl.pallas_call(
        _a0_body,
        grid=(2 * NB_U,),
        in_specs=[_rows(NCOLS), _wspec(NCOLS, HID), _bspec(HID),
                  _wspec(NCOLS, HID), _bspec(HID)],
        out_specs=[_rows(HID), _rows(HALF), _rows(HALF)],
        out_shape=[_sds(NTOT, HID), _sds(NTOT, HALF), _sds(NTOT, HALF)],
    )(x, ws, bs, wm, bm)


def _mid_call(su, ml, mr, ws, bs, wm):
    return pl.pallas_call(
        _mid_body,
        grid=(2 * NB_U,),
        in_specs=[_rows(HID), _rows(HALF), _rows(HALF),
                  _wspec(HID, HID), _bspec(HID), _wspec(HID, HID)],
        out_specs=[_rows(HID), _rows(HALF), _rows(HALF)],
        out_shape=[_sds(NTOT, HID), _sds(NTOT, HALF), _sds(NTOT, HALF)],
    )(su, ml, mr, ws, bs, wm)


def _fuse_call(su, ml, mr, w, b, off, outc):
    return pl.pallas_call(
        _fuse_body,
        grid=(NB_U,),
        in_specs=[_rows(HID, off), _rows(HALF, off), _rows(HALF, off),
                  _wfix(HID, outc), _bfix(outc)],
        out_specs=_rows(outc),
        out_shape=_sds(NP, outc),
    )(su, ml, mr, w, b)


def _fuse_msg_call(su, ml, mr, w, off):
    return pl.pallas_call(
        _fuse_msg_body,
        grid=(NB_U,),
        in_specs=[_rows(HID, off), _rows(HALF, off), _rows(HALF, off),
                  _wfix(HID, HID)],
        out_specs=[_rows(HALF), _rows(HALF)],
        out_shape=[_sds(NP, HALF), _sds(NP, HALF)],
    )(su, ml, mr, w)


def _make_seg(nphase):
    out_rows = nphase * NP
    mesh = plsc.VectorSubcoreMesh(core_axis_name="c", subcore_axis_name="s")

    @functools.partial(
        pl.kernel,
        out_type=(jax.ShapeDtypeStruct((out_rows, HALF), f32),
                  jax.ShapeDtypeStruct((out_rows, HALF), f32)),
        mesh=mesh,
        scratch_types=(
            pltpu.VMEM((2, WIN, CHUNK), i32),
            pltpu.VMEM((2, WIN, CHUNK), i32),
            pltpu.VMEM((NBUF, CHUNK, HALF), f32),
            pltpu.VMEM_SHARED((ACC_ROWS, HALF), f32),
            pltpu.SemaphoreType.DMA((NBUF,)),
            pltpu.SemaphoreType.DMA((NBUF,)),
        ),
    )
    def seg(ml, mr, srcs, dsts, zeros, outl, outr, srcv, dstv, bufs, accum,
            gsem, ssem):
        c = lax.axis_index("c")
        s = lax.axis_index("s")

        def run(tab, out):
            for p in range(nphase):
                base = p * CPP + s * CPS
                pltpu.sync_copy(zeros, accum.at[pl.ds(s * ZROWS, ZROWS), :])
                pltpu.sync_copy(srcs.at[pl.ds(base, WIN), :], srcv.at[0])
                pltpu.sync_copy(dsts.at[pl.ds(base, WIN), :], dstv.at[0])
                pltpu.sync_copy(srcs.at[pl.ds(base + WIN, WIN), :], srcv.at[1])
                pltpu.sync_copy(dsts.at[pl.ds(base + WIN, WIN), :], dstv.at[1])
                plsc.subcore_barrier()

                def gather_chunk(sl, row, b):
                    for h in range(2):
                        hs = pl.ds(h * 64, 64)
                        pltpu.async_copy(tab.at[srcv.at[sl, row, hs]],
                                         bufs.at[b, hs, :], gsem.at[b])

                for b in range(NBUF):
                    gather_chunk(0, b, b)

                def group(i0, _):
                    g0 = i0 * NBUF
                    w = g0 // WIN

                    @pl.when((lax.rem(g0, WIN) == 0) & (w >= 1)
                             & (w + 1 < NWIN))
                    def _():
                        sl = lax.rem(w + 1, 2)
                        pltpu.sync_copy(
                            srcs.at[pl.ds(base + (w + 1) * WIN, WIN), :],
                            srcv.at[sl])
                        pltpu.sync_copy(
                            dsts.at[pl.ds(base + (w + 1) * WIN, WIN), :],
                            dstv.at[sl])

                    for b in range(NBUF):
                        g = g0 + b
                        sl = lax.rem(g // WIN, 2)
                        row = lax.rem(g, WIN)
                        pltpu.make_async_copy(tab.at[srcv.at[sl, row]],
                                              bufs.at[b], gsem.at[b]).wait()
                        pltpu.async_copy(bufs.at[b],
                                         accum.at[dstv.at[sl, row]],
                                         ssem.at[b], add=True)
                    for b in range(NBUF):
                        g = g0 + b
                        g2 = g + NBUF
                        sl = lax.rem(g // WIN, 2)
                        row = lax.rem(g, WIN)
                        sl2 = lax.rem(g2 // WIN, 2)
                        row2 = lax.rem(g2, WIN)
                        pltpu.make_async_copy(bufs.at[b],
                                              accum.at[dstv.at[sl, row]],
                                              ssem.at[b]).wait()

                        @pl.when(g2 < CPS)
                        def _():
                            gather_chunk(sl2, row2, b)
                    return 0

                lax.fori_loop(0, CPS // NBUF, group, 0)
                plsc.subcore_barrier()
                pltpu.sync_copy(accum.at[pl.ds(s * OROWS, OROWS), :],
                                out.at[pl.ds(p * NP + s * OROWS, OROWS), :])
                plsc.subcore_barrier()

        @pl.when(c == 0)
        def _():
            run(ml, outl)

        @pl.when(c == 1)
        def _():
            run(mr, outr)

    return seg


def kernel(x_user, x_item, edge_u2i, edge_i2u, Wemb_u, bemb_u, Wemb_i, bemb_i,
           Wself_u, Wself_i, bias_u, bias_i, Wmsg_u2i, Wmsg_i2u, Wmlp, bmlp):
    eye = jnp.eye(NCOLS, dtype=f32)
    Wbig_u = (eye[:, :, None] * Wemb_u[None]).reshape(NCOLS, HID)
    Wbig_i = (eye[:, :, None] * Wemb_i[None]).reshape(NCOLS, HID)
    bflat_u = bemb_u.reshape(HID)
    bflat_i = bemb_i.reshape(HID)
    WsF = jnp.stack([Wbig_u @ Wself_u[0], Wbig_i @ Wself_i[0]])
    bsF = jnp.stack([bflat_u @ Wself_u[0] + bias_u[0],
                     bflat_i @ Wself_i[0] + bias_i[0]])[:, None, :]
    WmF = jnp.stack([Wbig_u @ Wmsg_u2i[0], Wbig_i @ Wmsg_i2u[0]])
    bmF = jnp.stack([bflat_u @ Wmsg_u2i[0], bflat_i @ Wmsg_i2u[0]])[:, None, :]
    Ws1 = jnp.stack([Wself_u[1], Wself_i[1]])
    bs1 = jnp.stack([bias_u[1], bias_i[1]])[:, None, :]
    Wm1 = jnp.stack([Wmsg_u2i[1], Wmsg_i2u[1]])

    src_u2i = edge_u2i[0].astype(i32)
    dst_u2i = edge_u2i[1].astype(i32)
    src_i2u = edge_i2u[0].astype(i32)
    dst_i2u = edge_i2u[1].astype(i32)
    def _sort_by_src(srca, dsta):
        key = jnp.sort(srca * 16384 + dsta, stable=False)
        return key >> 14, key & 16383
    src_i2u, dst_i2u = _sort_by_src(src_i2u, dst_i2u)
    src_u2i, dst_u2i = _sort_by_src(src_u2i, dst_u2i)
    npad = EPAD - E
    pad_s = (jnp.arange(npad, dtype=i32) * 13) % NU
    pad_d = NU + 64 + (jnp.arange(npad, dtype=i32) % 128)
    SRC2 = jnp.concatenate([src_i2u + NP, pad_s, src_u2i, pad_s]).reshape(2 * CPP, CHUNK)
    DST2 = jnp.concatenate([dst_i2u, pad_d, dst_u2i, pad_d]).reshape(2 * CPP, CHUNK)
    SRC1 = jnp.concatenate([src_i2u, pad_s]).reshape(CPP, CHUNK)
    DST1 = jnp.concatenate([dst_i2u, pad_d]).reshape(CPP, CHUNK)
    zeros = jnp.zeros((ZROWS, HALF), f32)

    X = jnp.concatenate([jnp.pad(x_user, ((0, NP - NU), (0, 0))),
                         jnp.pad(x_item, ((0, NP - NI), (0, 0)))], axis=0)

    SU, ML, MR = _a0_call(X, WsF, bsF, WmF, bmF)
    seg2 = _make_seg(2)
    MSGL, MSGR = seg2(ML, MR, SRC2, DST2, zeros)
    SU, ML, MR = _mid_call(SU, MSGL, MSGR, Ws1, bs1, Wm1)
    MSGL, MSGR = seg2(ML, MR, SRC2, DST2, zeros)
    SU2 = _fuse_call(SU, MSGL, MSGR, Wself_u[2][None], bias_u[2][None, None], 0, HID)
    M2L, M2R = _fuse_msg_call(SU, MSGL, MSGR, Wmsg_i2u[2][None], NB_U)
    seg1 = _make_seg(1)
    MSG2L, MSG2R = seg1(M2L, M2R, SRC1, DST1, zeros)
    out = _fuse_call(SU2, MSG2L, MSG2R, Wmlp[None], bmlp[None, None], 0, OUTD)
    return out[:NU]


# speedup vs baseline: 3.2378x; 1.0313x over previous
import functools

import jax
import jax.numpy as jnp
from jax import lax
from jax.experimental import pallas as pl
from jax.experimental.pallas import tpu as pltpu
from jax.experimental.pallas import tpu_sc as plsc

f32 = jnp.float32
i32 = jnp.int32

NU = 10000
NI = 10000
NP = 10240
NTOT = 2 * NP
E = 160000
NCOLS = 4
HID = 256
HALF = 128
OUTD = 64

NS = 16
CHUNK = 64
CPP = 2560
EPAD = CPP * CHUNK
CPS = CPP // NS
ACC_ROWS = NP
TRASH = 10100
ZROWS = ACC_ROWS // NS
OROWS = ZROWS
NBUF = 4
WIN = 16
NWIN = CPS // WIN

RB = 640
NB_U = NP // RB


def _a0_body(x_ref, ws_ref, bs_ref, wm_ref, bm_ref, su_ref, ml_ref, mr_ref):
    x = x_ref[...]
    su_ref[...] = jnp.dot(x, ws_ref[0], preferred_element_type=f32) + bs_ref[0]
    m = jnp.dot(x, wm_ref[0], preferred_element_type=f32) + bm_ref[0]
    ml_ref[...] = m[:, :HALF]
    mr_ref[...] = m[:, HALF:]


def _mid_body(su_ref, ml_ref, mr_ref, ws_ref, bs_ref, wm_ref, su_o, ml_o, mr_o):
    h = su_ref[...] + jnp.concatenate([ml_ref[...], mr_ref[...]], axis=1)
    h = jnp.maximum(h, 0.0)
    su_o[...] = jnp.dot(h, ws_ref[0], preferred_element_type=f32) + bs_ref[0]
    m = jnp.dot(h, wm_ref[0], preferred_element_type=f32)
    ml_o[...] = m[:, :HALF]
    mr_o[...] = m[:, HALF:]


def _fuse_body(su_ref, ml_ref, mr_ref, w_ref, b_ref, o_ref):
    h = su_ref[...] + jnp.concatenate([ml_ref[...], mr_ref[...]], axis=1)
    h = jnp.maximum(h, 0.0)
    o_ref[...] = jnp.dot(h, w_ref[0], preferred_element_type=f32) + b_ref[0]


def _fuse_msg_body(su_ref, ml_ref, mr_ref, w_ref, ml_o, mr_o):
    h = su_ref[...] + jnp.concatenate([ml_ref[...], mr_ref[...]], axis=1)
    h = jnp.maximum(h, 0.0)
    m = jnp.dot(h, w_ref[0], preferred_element_type=f32)
    ml_o[...] = m[:, :HALF]
    mr_o[...] = m[:, HALF:]


def _rows(cols, off=0):
    return pl.BlockSpec((RB, cols), lambda i, o=off: (i + o, 0))


def _wspec(a, b):
    return pl.BlockSpec((1, a, b), lambda i: (i // NB_U, 0, 0))


def _bspec(b):
    return pl.BlockSpec((1, 1, b), lambda i: (i // NB_U, 0, 0))


def _wfix(a, b):
    return pl.BlockSpec((1, a, b), lambda i: (0, 0, 0))


def _bfix(b):
    return pl.BlockSpec((1, 1, b), lambda i: (0, 0, 0))


def _sds(r, c):
    return jax.ShapeDtypeStruct((r, c), f32)


def _a0_call(x, ws, bs, wm, bm):
    return pl.pallas_call(
        _a0_body,
        grid=(2 * NB_U,),
        in_specs=[_rows(NCOLS), _wspec(NCOLS, HID), _bspec(HID),
                  _wspec(NCOLS, HID), _bspec(HID)],
        out_specs=[_rows(HID), _rows(HALF), _rows(HALF)],
        out_shape=[_sds(NTOT, HID), _sds(NTOT, HALF), _sds(NTOT, HALF)],
    )(x, ws, bs, wm, bm)


def _mid_call(su, ml, mr, ws, bs, wm):
    return pl.pallas_call(
        _mid_body,
        grid=(2 * NB_U,),
        in_specs=[_rows(HID), _rows(HALF), _rows(HALF),
                  _wspec(HID, HID), _bspec(HID), _wspec(HID, HID)],
        out_specs=[_rows(HID), _rows(HALF), _rows(HALF)],
        out_shape=[_sds(NTOT, HID), _sds(NTOT, HALF), _sds(NTOT, HALF)],
    )(su, ml, mr, ws, bs, wm)


def _fuse_call(su, ml, mr, w, b, off, outc):
    return pl.pallas_call(
        _fuse_body,
        grid=(NB_U,),
        in_specs=[_rows(HID, off), _rows(HALF, off), _rows(HALF, off),
                  _wfix(HID, outc), _bfix(outc)],
        out_specs=_rows(outc),
        out_shape=_sds(NP, outc),
    )(su, ml, mr, w, b)


def _fuse_msg_call(su, ml, mr, w, off):
    return pl.pallas_call(
        _fuse_msg_body,
        grid=(NB_U,),
        in_specs=[_rows(HID, off), _rows(HALF, off), _rows(HALF, off),
                  _wfix(HID, HID)],
        out_specs=[_rows(HALF), _rows(HALF)],
        out_shape=[_sds(NP, HALF), _sds(NP, HALF)],
    )(su, ml, mr, w)


def _make_seg(nphase):
    out_rows = nphase * NP
    mesh = plsc.VectorSubcoreMesh(core_axis_name="c", subcore_axis_name="s")

    @functools.partial(
        pl.kernel,
        out_type=(jax.ShapeDtypeStruct((out_rows, HALF), f32),
                  jax.ShapeDtypeStruct((out_rows, HALF), f32)),
        mesh=mesh,
        scratch_types=(
            pltpu.VMEM((2, WIN, CHUNK), i32),
            pltpu.VMEM((2, WIN, CHUNK), i32),
            pltpu.VMEM((NBUF, CHUNK, HALF), f32),
            pltpu.VMEM_SHARED((ACC_ROWS, HALF), f32),
            pltpu.SemaphoreType.DMA((NBUF,)),
            pltpu.SemaphoreType.DMA((NBUF,)),
        ),
    )
    def seg(ml, mr, srcs, dsts, zeros, outl, outr, srcv, dstv, bufs, accum,
            gsem, ssem):
        c = lax.axis_index("c")
        s = lax.axis_index("s")

        def run(tab, out):
            for p in range(nphase):
                base = p * CPP + s * CPS
                pltpu.sync_copy(zeros, accum.at[pl.ds(s * ZROWS, ZROWS), :])
                pltpu.sync_copy(srcs.at[pl.ds(base, WIN), :], srcv.at[0])
                pltpu.sync_copy(dsts.at[pl.ds(base, WIN), :], dstv.at[0])
                pltpu.sync_copy(srcs.at[pl.ds(base + WIN, WIN), :], srcv.at[1])
                pltpu.sync_copy(dsts.at[pl.ds(base + WIN, WIN), :], dstv.at[1])
                plsc.subcore_barrier()

                def gather_chunk(sl, row, b):
                    pltpu.async_copy(tab.at[srcv.at[sl, row]],
                                     bufs.at[b], gsem.at[b])

                for b in range(NBUF):
                    gather_chunk(0, b, b)

                def group(i0, _):
                    g0 = i0 * NBUF
                    w = g0 // WIN

                    @pl.when((lax.rem(g0, WIN) == 0) & (w >= 1)
                             & (w + 1 < NWIN))
                    def _():
                        sl = lax.rem(w + 1, 2)
                        pltpu.sync_copy(
                            srcs.at[pl.ds(base + (w + 1) * WIN, WIN), :],
                            srcv.at[sl])
                        pltpu.sync_copy(
                            dsts.at[pl.ds(base + (w + 1) * WIN, WIN), :],
                            dstv.at[sl])

                    for b in range(NBUF):
                        g = g0 + b
                        sl = lax.rem(g // WIN, 2)
                        row = lax.rem(g, WIN)
                        pltpu.make_async_copy(tab.at[srcv.at[sl, row]],
                                              bufs.at[b], gsem.at[b]).wait()
                        pltpu.async_copy(bufs.at[b],
                                         accum.at[dstv.at[sl, row]],
                                         ssem.at[b], add=True)
                    for b in range(NBUF):
                        g = g0 + b
                        g2 = g + NBUF
                        sl = lax.rem(g // WIN, 2)
                        row = lax.rem(g, WIN)
                        sl2 = lax.rem(g2 // WIN, 2)
                        row2 = lax.rem(g2, WIN)
                        pltpu.make_async_copy(bufs.at[b],
                                              accum.at[dstv.at[sl, row]],
                                              ssem.at[b]).wait()

                        @pl.when(g2 < CPS)
                        def _():
                            gather_chunk(sl2, row2, b)
                    return 0

                lax.fori_loop(0, CPS // NBUF, group, 0)
                plsc.subcore_barrier()
                pltpu.sync_copy(accum.at[pl.ds(s * OROWS, OROWS), :],
                                out.at[pl.ds(p * NP + s * OROWS, OROWS), :])
                plsc.subcore_barrier()

        @pl.when(c == 0)
        def _():
            run(ml, outl)

        @pl.when(c == 1)
        def _():
            run(mr, outr)

    return seg


def kernel(x_user, x_item, edge_u2i, edge_i2u, Wemb_u, bemb_u, Wemb_i, bemb_i,
           Wself_u, Wself_i, bias_u, bias_i, Wmsg_u2i, Wmsg_i2u, Wmlp, bmlp):
    eye = jnp.eye(NCOLS, dtype=f32)
    Wbig_u = (eye[:, :, None] * Wemb_u[None]).reshape(NCOLS, HID)
    Wbig_i = (eye[:, :, None] * Wemb_i[None]).reshape(NCOLS, HID)
    bflat_u = bemb_u.reshape(HID)
    bflat_i = bemb_i.reshape(HID)
    WsF = jnp.stack([Wbig_u @ Wself_u[0], Wbig_i @ Wself_i[0]])
    bsF = jnp.stack([bflat_u @ Wself_u[0] + bias_u[0],
                     bflat_i @ Wself_i[0] + bias_i[0]])[:, None, :]
    WmF = jnp.stack([Wbig_u @ Wmsg_u2i[0], Wbig_i @ Wmsg_i2u[0]])
    bmF = jnp.stack([bflat_u @ Wmsg_u2i[0], bflat_i @ Wmsg_i2u[0]])[:, None, :]
    Ws1 = jnp.stack([Wself_u[1], Wself_i[1]])
    bs1 = jnp.stack([bias_u[1], bias_i[1]])[:, None, :]
    Wm1 = jnp.stack([Wmsg_u2i[1], Wmsg_i2u[1]])

    src_u2i = edge_u2i[0].astype(i32)
    dst_u2i = edge_u2i[1].astype(i32)
    src_i2u = edge_i2u[0].astype(i32)
    dst_i2u = edge_i2u[1].astype(i32)
    def _sort_by_src(srca, dsta):
        key = jnp.sort(srca * 16384 + dsta, stable=False)
        return key >> 14, key & 16383
    src_i2u, dst_i2u = _sort_by_src(src_i2u, dst_i2u)
    src_u2i, dst_u2i = _sort_by_src(src_u2i, dst_u2i)
    npad = EPAD - E
    pad_s = (jnp.arange(npad, dtype=i32) * 13) % NU
    pad_d = NU + 64 + (jnp.arange(npad, dtype=i32) % 128)
    SRC2 = jnp.concatenate([src_i2u + NP, pad_s, src_u2i, pad_s]).reshape(2 * CPP, CHUNK)
    DST2 = jnp.concatenate([dst_i2u, pad_d, dst_u2i, pad_d]).reshape(2 * CPP, CHUNK)
    SRC1 = jnp.concatenate([src_i2u, pad_s]).reshape(CPP, CHUNK)
    DST1 = jnp.concatenate([dst_i2u, pad_d]).reshape(CPP, CHUNK)
    zeros = jnp.zeros((ZROWS, HALF), f32)

    X = jnp.concatenate([jnp.pad(x_user, ((0, NP - NU), (0, 0))),
                         jnp.pad(x_item, ((0, NP - NI), (0, 0)))], axis=0)

    SU, ML, MR = _a0_call(X, WsF, bsF, WmF, bmF)
    seg2 = _make_seg(2)
    MSGL, MSGR = seg2(ML, MR, SRC2, DST2, zeros)
    SU, ML, MR = _mid_call(SU, MSGL, MSGR, Ws1, bs1, Wm1)
    MSGL, MSGR = seg2(ML, MR, SRC2, DST2, zeros)
    SU2 = _fuse_call(SU, MSGL, MSGR, Wself_u[2][None], bias_u[2][None, None], 0, HID)
    M2L, M2R = _fuse_msg_call(SU, MSGL, MSGR, Wmsg_i2u[2][None], NB_U)
    seg1 = _make_seg(1)
    MSG2L, MSG2R = seg1(M2L, M2R, SRC1, DST1, zeros)
    out = _fuse_call(SU2, MSG2L, MSG2R, Wmlp[None], bmlp[None, None], 0, OUTD)
    return out[:NU]


# speedup vs baseline: 3.4175x; 1.0555x over previous
import functools

import jax
import jax.numpy as jnp
from jax import lax
from jax.experimental import pallas as pl
from jax.experimental.pallas import tpu as pltpu
from jax.experimental.pallas import tpu_sc as plsc

f32 = jnp.float32
i32 = jnp.int32

NU = 10000
NI = 10000
NP = 10240
NTOT = 2 * NP
E = 160000
NCOLS = 4
HID = 256
HALF = 128
OUTD = 64

NS = 16
CHUNK = 64
CPP = 2560
EPAD = CPP * CHUNK
CPS = CPP // NS
ACC_ROWS = NP
TRASH = 10100
ZROWS = ACC_ROWS // NS
OROWS = ZROWS
NBUF = 5
WIN = 16
NWIN = CPS // WIN

RB = 640
NB_U = NP // RB


def _a0_body(x_ref, ws_ref, bs_ref, wm_ref, bm_ref, su_ref, ml_ref, mr_ref):
    x = x_ref[...]
    su_ref[...] = jnp.dot(x, ws_ref[0], preferred_element_type=f32) + bs_ref[0]
    m = jnp.dot(x, wm_ref[0], preferred_element_type=f32) + bm_ref[0]
    ml_ref[...] = m[:, :HALF]
    mr_ref[...] = m[:, HALF:]


def _mid_body(su_ref, ml_ref, mr_ref, ws_ref, bs_ref, wm_ref, su_o, ml_o, mr_o):
    h = su_ref[...] + jnp.concatenate([ml_ref[...], mr_ref[...]], axis=1)
    h = jnp.maximum(h, 0.0)
    su_o[...] = jnp.dot(h, ws_ref[0], preferred_element_type=f32) + bs_ref[0]
    m = jnp.dot(h, wm_ref[0], preferred_element_type=f32)
    ml_o[...] = m[:, :HALF]
    mr_o[...] = m[:, HALF:]


def _fuse_body(su_ref, ml_ref, mr_ref, w_ref, b_ref, o_ref):
    h = su_ref[...] + jnp.concatenate([ml_ref[...], mr_ref[...]], axis=1)
    h = jnp.maximum(h, 0.0)
    o_ref[...] = jnp.dot(h, w_ref[0], preferred_element_type=f32) + b_ref[0]


def _fuse_msg_body(su_ref, ml_ref, mr_ref, w_ref, ml_o, mr_o):
    h = su_ref[...] + jnp.concatenate([ml_ref[...], mr_ref[...]], axis=1)
    h = jnp.maximum(h, 0.0)
    m = jnp.dot(h, w_ref[0], preferred_element_type=f32)
    ml_o[...] = m[:, :HALF]
    mr_o[...] = m[:, HALF:]


def _rows(cols, off=0):
    return pl.BlockSpec((RB, cols), lambda i, o=off: (i + o, 0))


def _wspec(a, b):
    return pl.BlockSpec((1, a, b), lambda i: (i // NB_U, 0, 0))


def _bspec(b):
    return pl.BlockSpec((1, 1, b), lambda i: (i // NB_U, 0, 0))


def _wfix(a, b):
    return pl.BlockSpec((1, a, b), lambda i: (0, 0, 0))


def _bfix(b):
    return pl.BlockSpec((1, 1, b), lambda i: (0, 0, 0))


def _sds(r, c):
    return jax.ShapeDtypeStruct((r, c), f32)


def _a0_call(x, ws, bs, wm, bm):
    return pl.pallas_call(
        _a0_body,
        grid=(2 * NB_U,),
        in_specs=[_rows(NCOLS), _wspec(NCOLS, HID), _bspec(HID),
                  _wspec(NCOLS, HID), _bspec(HID)],
        out_specs=[_rows(HID), _rows(HALF), _rows(HALF)],
        out_shape=[_sds(NTOT, HID), _sds(NTOT, HALF), _sds(NTOT, HALF)],
    )(x, ws, bs, wm, bm)


def _mid_call(su, ml, mr, ws, bs, wm):
    return pl.pallas_call(
        _mid_body,
        grid=(2 * NB_U,),
        in_specs=[_rows(HID), _rows(HALF), _rows(HALF),
                  _wspec(HID, HID), _bspec(HID), _wspec(HID, HID)],
        out_specs=[_rows(HID), _rows(HALF), _rows(HALF)],
        out_shape=[_sds(NTOT, HID), _sds(NTOT, HALF), _sds(NTOT, HALF)],
    )(su, ml, mr, ws, bs, wm)


def _fuse_call(su, ml, mr, w, b, off, outc):
    return pl.pallas_call(
        _fuse_body,
        grid=(NB_U,),
        in_specs=[_rows(HID, off), _rows(HALF, off), _rows(HALF, off),
                  _wfix(HID, outc), _bfix(outc)],
        out_specs=_rows(outc),
        out_shape=_sds(NP, outc),
    )(su, ml, mr, w, b)


def _fuse_msg_call(su, ml, mr, w, off):
    return pl.pallas_call(
        _fuse_msg_body,
        grid=(NB_U,),
        in_specs=[_rows(HID, off), _rows(HALF, off), _rows(HALF, off),
                  _wfix(HID, HID)],
        out_specs=[_rows(HALF), _rows(HALF)],
        out_shape=[_sds(NP, HALF), _sds(NP, HALF)],
    )(su, ml, mr, w)


def _make_seg(nphase):
    out_rows = nphase * NP
    mesh = plsc.VectorSubcoreMesh(core_axis_name="c", subcore_axis_name="s")

    @functools.partial(
        pl.kernel,
        out_type=(jax.ShapeDtypeStruct((out_rows, HALF), f32),
                  jax.ShapeDtypeStruct((out_rows, HALF), f32)),
        mesh=mesh,
        scratch_types=(
            pltpu.VMEM((2, WIN, CHUNK), i32),
            pltpu.VMEM((2, WIN, CHUNK), i32),
            pltpu.VMEM((NBUF, CHUNK, HALF), f32),
            pltpu.VMEM_SHARED((ACC_ROWS, HALF), f32),
            pltpu.SemaphoreType.DMA((NBUF,)),
            pltpu.SemaphoreType.DMA((NBUF,)),
        ),
    )
    def seg(ml, mr, srcs, dsts, zeros, outl, outr, srcv, dstv, bufs, accum,
            gsem, ssem):
        c = lax.axis_index("c")
        s = lax.axis_index("s")

        def run(tab, out):
            for p in range(nphase):
                base = p * CPP + s * CPS
                pltpu.sync_copy(zeros, accum.at[pl.ds(s * ZROWS, ZROWS), :])
                pltpu.sync_copy(srcs.at[pl.ds(base, WIN), :], srcv.at[0])
                pltpu.sync_copy(dsts.at[pl.ds(base, WIN), :], dstv.at[0])
                pltpu.sync_copy(srcs.at[pl.ds(base + WIN, WIN), :], srcv.at[1])
                pltpu.sync_copy(dsts.at[pl.ds(base + WIN, WIN), :], dstv.at[1])
                plsc.subcore_barrier()

                def gather_chunk(sl, row, b):
                    pltpu.async_copy(tab.at[srcv.at[sl, row]],
                                     bufs.at[b], gsem.at[b])

                for b in range(NBUF):
                    gather_chunk(0, b, b)

                def group(i0, _):
                    g0 = i0 * NBUF
                    w = g0 // WIN

                    @pl.when((lax.rem(g0, WIN) == 0) & (w >= 1)
                             & (w + 1 < NWIN))
                    def _():
                        sl = lax.rem(w + 1, 2)
                        pltpu.sync_copy(
                            srcs.at[pl.ds(base + (w + 1) * WIN, WIN), :],
                            srcv.at[sl])
                        pltpu.sync_copy(
                            dsts.at[pl.ds(base + (w + 1) * WIN, WIN), :],
                            dstv.at[sl])

                    for b in range(NBUF):
                        g = g0 + b
                        sl = lax.rem(g // WIN, 2)
                        row = lax.rem(g, WIN)
                        pltpu.make_async_copy(tab.at[srcv.at[sl, row]],
                                              bufs.at[b], gsem.at[b]).wait()
                        pltpu.async_copy(bufs.at[b],
                                         accum.at[dstv.at[sl, row]],
                                         ssem.at[b], add=True)
                    for b in range(NBUF):
                        g = g0 + b
                        g2 = g + NBUF
                        sl = lax.rem(g // WIN, 2)
                        row = lax.rem(g, WIN)
                        sl2 = lax.rem(g2 // WIN, 2)
                        row2 = lax.rem(g2, WIN)
                        pltpu.make_async_copy(bufs.at[b],
                                              accum.at[dstv.at[sl, row]],
                                              ssem.at[b]).wait()

                        @pl.when(g2 < CPS)
                        def _():
                            gather_chunk(sl2, row2, b)
                    return 0

                lax.fori_loop(0, CPS // NBUF, group, 0)
                plsc.subcore_barrier()
                pltpu.sync_copy(accum.at[pl.ds(s * OROWS, OROWS), :],
                                out.at[pl.ds(p * NP + s * OROWS, OROWS), :])
                plsc.subcore_barrier()

        @pl.when(c == 0)
        def _():
            run(ml, outl)

        @pl.when(c == 1)
        def _():
            run(mr, outr)

    return seg


def kernel(x_user, x_item, edge_u2i, edge_i2u, Wemb_u, bemb_u, Wemb_i, bemb_i,
           Wself_u, Wself_i, bias_u, bias_i, Wmsg_u2i, Wmsg_i2u, Wmlp, bmlp):
    eye = jnp.eye(NCOLS, dtype=f32)
    Wbig_u = (eye[:, :, None] * Wemb_u[None]).reshape(NCOLS, HID)
    Wbig_i = (eye[:, :, None] * Wemb_i[None]).reshape(NCOLS, HID)
    bflat_u = bemb_u.reshape(HID)
    bflat_i = bemb_i.reshape(HID)
    WsF = jnp.stack([Wbig_u @ Wself_u[0], Wbig_i @ Wself_i[0]])
    bsF = jnp.stack([bflat_u @ Wself_u[0] + bias_u[0],
                     bflat_i @ Wself_i[0] + bias_i[0]])[:, None, :]
    WmF = jnp.stack([Wbig_u @ Wmsg_u2i[0], Wbig_i @ Wmsg_i2u[0]])
    bmF = jnp.stack([bflat_u @ Wmsg_u2i[0], bflat_i @ Wmsg_i2u[0]])[:, None, :]
    Ws1 = jnp.stack([Wself_u[1], Wself_i[1]])
    bs1 = jnp.stack([bias_u[1], bias_i[1]])[:, None, :]
    Wm1 = jnp.stack([Wmsg_u2i[1], Wmsg_i2u[1]])

    src_u2i = edge_u2i[0].astype(i32)
    dst_u2i = edge_u2i[1].astype(i32)
    src_i2u = edge_i2u[0].astype(i32)
    dst_i2u = edge_i2u[1].astype(i32)
    def _sort_by_src(srca, dsta):
        key = jnp.sort(srca * 16384 + dsta, stable=False)
        return key >> 14, key & 16383
    src_i2u, dst_i2u = _sort_by_src(src_i2u, dst_i2u)
    src_u2i, dst_u2i = _sort_by_src(src_u2i, dst_u2i)
    npad = EPAD - E
    pad_s = (jnp.arange(npad, dtype=i32) * 13) % NU
    pad_d = NU + 64 + (jnp.arange(npad, dtype=i32) % 128)
    SRC2 = jnp.concatenate([src_i2u + NP, pad_s, src_u2i, pad_s]).reshape(2 * CPP, CHUNK)
    DST2 = jnp.concatenate([dst_i2u, pad_d, dst_u2i, pad_d]).reshape(2 * CPP, CHUNK)
    SRC1 = jnp.concatenate([src_i2u, pad_s]).reshape(CPP, CHUNK)
    DST1 = jnp.concatenate([dst_i2u, pad_d]).reshape(CPP, CHUNK)
    zeros = jnp.zeros((ZROWS, HALF), f32)

    X = jnp.concatenate([jnp.pad(x_user, ((0, NP - NU), (0, 0))),
                         jnp.pad(x_item, ((0, NP - NI), (0, 0)))], axis=0)

    SU, ML, MR = _a0_call(X, WsF, bsF, WmF, bmF)
    seg2 = _make_seg(2)
    MSGL, MSGR = seg2(ML, MR, SRC2, DST2, zeros)
    SU, ML, MR = _mid_call(SU, MSGL, MSGR, Ws1, bs1, Wm1)
    MSGL, MSGR = seg2(ML, MR, SRC2, DST2, zeros)
    SU2 = _fuse_call(SU, MSGL, MSGR, Wself_u[2][None], bias_u[2][None, None], 0, HID)
    M2L, M2R = _fuse_msg_call(SU, MSGL, MSGR, Wmsg_i2u[2][None], NB_U)
    seg1 = _make_seg(1)
    MSG2L, MSG2R = seg1(M2L, M2R, SRC1, DST1, zeros)
    out = _fuse_call(SU2, MSG2L, MSG2R, Wmlp[None], bmlp[None, None], 0, OUTD)
    return out[:NU]
